# initial kernel scaffold (unmeasured)
import jax
import jax.numpy as jnp
from jax import lax
from jax.experimental import pallas as pl
from jax.experimental.pallas import tpu as pltpu

N_DEV = 8


def kernel(x, w_mat):
    m, k = x.shape
    n = w_mat.shape[1]
    blk = n // N_DEV

    def body(x_ref, w_ref, out_ref, y_ref, q_ref, recv_ref,
             amax_send_ref, amax_all_ref,
             data_send_sems, data_recv_sems, amax_send_sems, amax_recv_sems):
        my = lax.axis_index("i")

        y_ref[...] = jnp.dot(
            x_ref[...], w_ref[...], preferred_element_type=jnp.float32
        )

        local_amax = jnp.max(jnp.abs(y_ref[...]))
        amax_send_ref[...] = jnp.full((1, 128), local_amax, jnp.float32)
        amax_all_ref[pl.ds(0, 1), :] = amax_send_ref[...]

        amax_rdmas = []
        for t in range(1, N_DEV):
            peer = lax.rem(my + t, N_DEV)
            rdma = pltpu.make_async_remote_copy(
                src_ref=amax_send_ref,
                dst_ref=amax_all_ref.at[pl.ds(t, 1), :],
                send_sem=amax_send_sems.at[t],
                recv_sem=amax_recv_sems.at[t],
                device_id=(peer,),
                device_id_type=pl.DeviceIdType.MESH,
            )
            rdma.start()
            amax_rdmas.append(rdma)
        for rdma in amax_rdmas:
            rdma.wait_recv()
        gmax = jnp.max(amax_all_ref[...])
        scale = gmax / 127.0
        inv_scale = 127.0 / gmax

        for t in range(N_DEV):
            j = lax.rem(my + t, N_DEV)
            yblk = y_ref[:, pl.ds(j * blk, blk)]
            q = jnp.clip(jnp.round(yblk * inv_scale), -127.0, 127.0)
            q_ref[t, :, :] = q.astype(jnp.int8)

        data_rdmas = []
        for t in range(1, N_DEV):
            peer = lax.rem(my + t, N_DEV)
            rdma = pltpu.make_async_remote_copy(
                src_ref=q_ref.at[t],
                dst_ref=recv_ref.at[t],
                send_sem=data_send_sems.at[t],
                recv_sem=data_recv_sems.at[t],
                device_id=(peer,),
                device_id_type=pl.DeviceIdType.MESH,
            )
            rdma.start()
            data_rdmas.append(rdma)

        out_ref[pl.ds(my * m, m), :] = q_ref[0, :, :].astype(jnp.float32) * scale

        for t in range(1, N_DEV):
            src = lax.rem(my + N_DEV - t, N_DEV)
            data_rdmas[t - 1].wait_recv()
            out_ref[pl.ds(src * m, m), :] = (
                recv_ref[t, :, :].astype(jnp.float32) * scale
            )

        for rdma in amax_rdmas:
            rdma.wait_send()
        for rdma in data_rdmas:
            rdma.wait_send()

    out_shape = jax.ShapeDtypeStruct((N_DEV * m, blk), jnp.float32)
    return pl.pallas_call(
        body,
        out_shape=out_shape,
        in_specs=[
            pl.BlockSpec(memory_space=pltpu.VMEM),
            pl.BlockSpec(memory_space=pltpu.VMEM),
        ],
        out_specs=pl.BlockSpec(memory_space=pltpu.VMEM),
        scratch_shapes=[
            pltpu.VMEM((m, n), jnp.float32),
            pltpu.VMEM((N_DEV, m, blk), jnp.int8),
            pltpu.VMEM((N_DEV, m, blk), jnp.int8),
            pltpu.VMEM((1, 128), jnp.float32),
            pltpu.VMEM((N_DEV, 128), jnp.float32),
            pltpu.SemaphoreType.DMA((N_DEV,)),
            pltpu.SemaphoreType.DMA((N_DEV,)),
            pltpu.SemaphoreType.DMA((N_DEV,)),
            pltpu.SemaphoreType.DMA((N_DEV,)),
        ],
        compiler_params=pltpu.CompilerParams(collective_id=0),
    )(x, w_mat)


# baseline (device time: 48145 ns/iter reference)
import os

import jax
import jax.numpy as jnp
from jax import lax
from jax.experimental import pallas as pl
from jax.experimental.pallas import tpu as pltpu

N_DEV = 8
STAGE = int(os.environ.get("KSTAGE", "4"))


def kernel(x, w_mat):
    m, k = x.shape
    n = w_mat.shape[1]
    blk = n // N_DEV

    def body(x_ref, w_ref, out_ref, y_ref, q_ref, recv_ref,
             amax_send_ref, amax_all_ref, copy_sem,
             data_send_sems, data_recv_sems, amax_send_sems, amax_recv_sems):
        my = lax.axis_index("i")

        y_ref[...] = jnp.dot(
            x_ref[...], w_ref[...], preferred_element_type=jnp.float32
        )
        local_amax = jnp.max(jnp.abs(y_ref[...]))

        if STAGE < 2:
            out_ref[pl.ds(0, m), :] = y_ref[:, pl.ds(0, blk)] * local_amax
            return

        amax_send_ref[...] = jnp.full((1, 128), local_amax, jnp.float32)
        amax_all_ref[pl.ds(0, 1), :] = amax_send_ref[...]

        amax_rdmas = []
        for t in range(1, N_DEV):
            peer = (my + t) % N_DEV
            rdma = pltpu.make_async_remote_copy(
                src_ref=amax_send_ref,
                dst_ref=amax_all_ref.at[pl.ds(t, 1), :],
                send_sem=amax_send_sems.at[t],
                recv_sem=amax_recv_sems.at[t],
                device_id=(peer,),
                device_id_type=pl.DeviceIdType.MESH,
            )
            rdma.start()
            amax_rdmas.append(rdma)
        for rdma in amax_rdmas:
            rdma.wait_recv()
        gmax = jnp.max(amax_all_ref[...])
        scale = gmax / 127.0
        inv_scale = 127.0 / gmax

        if STAGE < 3:
            out_ref[pl.ds(0, m), :] = y_ref[:, pl.ds(0, blk)] * inv_scale
            for rdma in amax_rdmas:
                rdma.wait_send()
            return

        q_ref[...] = jnp.clip(
            jnp.round(y_ref[...] * inv_scale), -127.0, 127.0
        ).astype(jnp.int8)

        if STAGE < 4:
            out_ref[pl.ds(0, m), :] = (
                q_ref[:, pl.ds(0, blk)].astype(jnp.float32) * scale
            )
            for rdma in amax_rdmas:
                rdma.wait_send()
            return

        data_rdmas = []
        for t in range(1, N_DEV):
            peer = (my + t) % N_DEV
            rdma = pltpu.make_async_remote_copy(
                src_ref=q_ref.at[:, pl.ds(peer * blk, blk)],
                dst_ref=recv_ref.at[t],
                send_sem=data_send_sems.at[t],
                recv_sem=data_recv_sems.at[t],
                device_id=(peer,),
                device_id_type=pl.DeviceIdType.MESH,
            )
            rdma.start()
            data_rdmas.append(rdma)

        own = pltpu.make_async_copy(
            q_ref.at[:, pl.ds(my * blk, blk)], recv_ref.at[0], copy_sem
        )
        own.start()
        own.wait()

        for t in range(N_DEV):
            src = (my + N_DEV - t) % N_DEV
            if t > 0:
                data_rdmas[t - 1].wait_recv()
            out_ref[pl.ds(src * m, m), :] = (
                recv_ref[t, :, :].astype(jnp.float32) * scale
            )

        for rdma in amax_rdmas:
            rdma.wait_send()
        for rdma in data_rdmas:
            rdma.wait_send()

    out_shape = jax.ShapeDtypeStruct((N_DEV * m, blk), jnp.float32)
    return pl.pallas_call(
        body,
        out_shape=out_shape,
        in_specs=[
            pl.BlockSpec(memory_space=pltpu.VMEM),
            pl.BlockSpec(memory_space=pltpu.VMEM),
        ],
        out_specs=pl.BlockSpec(memory_space=pltpu.VMEM),
        scratch_shapes=[
            pltpu.VMEM((m, n), jnp.float32),
            pltpu.VMEM((m, n), jnp.int8),
            pltpu.VMEM((N_DEV, m, blk), jnp.int8),
            pltpu.VMEM((1, 128), jnp.float32),
            pltpu.VMEM((N_DEV, 128), jnp.float32),
            pltpu.SemaphoreType.DMA,
            pltpu.SemaphoreType.DMA((N_DEV,)),
            pltpu.SemaphoreType.DMA((N_DEV,)),
            pltpu.SemaphoreType.DMA((N_DEV,)),
            pltpu.SemaphoreType.DMA((N_DEV,)),
        ],
        compiler_params=pltpu.CompilerParams(
            vmem_limit_bytes=60 * 1024 * 1024,
        ),
    )(x, w_mat)


# device time: 43277 ns/iter; 1.1125x vs baseline; 1.1125x over previous
import jax
import jax.numpy as jnp
from jax import lax
from jax.experimental import pallas as pl
from jax.experimental.pallas import tpu as pltpu

N_DEV = 8
CHUNK = 2


def kernel(x, w_mat):
    m, k = x.shape
    n = w_mat.shape[1]
    blk = n // N_DEV
    cw = CHUNK * blk
    n_chunks = N_DEV // CHUNK

    def body(x_ref, w_ref, out_ref, send_ref, recv_ref,
             amax_send_ref, amax_all_ref, copy_sem,
             data_send_sems, data_recv_sems, amax_send_sems, amax_recv_sems):
        my = lax.axis_index("i")

        barrier_sem = pltpu.get_barrier_semaphore()
        for t in range(1, N_DEV):
            pl.semaphore_signal(
                barrier_sem, inc=1,
                device_id=((my + t) % N_DEV,),
                device_id_type=pl.DeviceIdType.MESH,
            )

        local_amax = jnp.float32(0.0)
        for c in range(n_chunks):
            yc = jnp.dot(
                x_ref[...], w_ref[:, pl.ds(c * cw, cw)],
                preferred_element_type=jnp.float32,
            )
            local_amax = jnp.maximum(local_amax, jnp.max(jnp.abs(yc)))
            for u in range(CHUNK):
                j = c * CHUNK + u
                send_ref[j, :, :] = yc[:, u * blk:(u + 1) * blk].astype(
                    jnp.bfloat16
                )
            if c == 0:
                pl.semaphore_wait(barrier_sem, N_DEV - 1)
            for u in range(CHUNK):
                j = c * CHUNK + u
                t = (j - my) % N_DEV

                @pl.when(j != my)
                def _(j=j, t=t):
                    rdma = pltpu.make_async_remote_copy(
                        src_ref=send_ref.at[j],
                        dst_ref=recv_ref.at[t],
                        send_sem=data_send_sems.at[j],
                        recv_sem=data_recv_sems.at[t],
                        device_id=(j,),
                        device_id_type=pl.DeviceIdType.MESH,
                    )
                    rdma.start()

                @pl.when(j == my)
                def _(j=j):
                    cp = pltpu.make_async_copy(
                        send_ref.at[j], recv_ref.at[0], copy_sem
                    )
                    cp.start()
                    cp.wait()

        amax_send_ref[...] = jnp.full((1, 128), local_amax, jnp.float32)
        amax_all_ref[pl.ds(0, 1), :] = amax_send_ref[...]
        amax_rdmas = []
        for t in range(1, N_DEV):
            rdma = pltpu.make_async_remote_copy(
                src_ref=amax_send_ref,
                dst_ref=amax_all_ref.at[pl.ds(t, 1), :],
                send_sem=amax_send_sems.at[t],
                recv_sem=amax_recv_sems.at[t],
                device_id=((my + t) % N_DEV,),
                device_id_type=pl.DeviceIdType.MESH,
            )
            rdma.start()
            amax_rdmas.append(rdma)
        for rdma in amax_rdmas:
            rdma.wait_recv()
        gmax = jnp.max(amax_all_ref[...])
        scale = gmax / 127.0
        inv_scale = 127.0 / gmax

        for t in range(N_DEV):
            if t > 0:
                pltpu.make_async_remote_copy(
                    src_ref=send_ref.at[0],
                    dst_ref=recv_ref.at[t],
                    send_sem=data_send_sems.at[0],
                    recv_sem=data_recv_sems.at[t],
                    device_id=(0,),
                    device_id_type=pl.DeviceIdType.MESH,
                ).wait_recv()
            src = (my + N_DEV - t) % N_DEV
            q = jnp.clip(
                jnp.round(recv_ref[t, :, :].astype(jnp.float32) * inv_scale),
                -127.0, 127.0,
            )
            out_ref[pl.ds(src * m, m), :] = q * scale

        for j in range(N_DEV):
            @pl.when(j != my)
            def _(j=j):
                pltpu.make_async_remote_copy(
                    src_ref=send_ref.at[j],
                    dst_ref=recv_ref.at[0],
                    send_sem=data_send_sems.at[j],
                    recv_sem=data_recv_sems.at[0],
                    device_id=(0,),
                    device_id_type=pl.DeviceIdType.MESH,
                ).wait_send()
        for rdma in amax_rdmas:
            rdma.wait_send()

    out_shape = jax.ShapeDtypeStruct((N_DEV * m, blk), jnp.float32)
    return pl.pallas_call(
        body,
        out_shape=out_shape,
        in_specs=[
            pl.BlockSpec(memory_space=pltpu.VMEM),
            pl.BlockSpec(memory_space=pltpu.VMEM),
        ],
        out_specs=pl.BlockSpec(memory_space=pltpu.VMEM),
        scratch_shapes=[
            pltpu.VMEM((N_DEV, m, blk), jnp.bfloat16),
            pltpu.VMEM((N_DEV, m, blk), jnp.bfloat16),
            pltpu.VMEM((1, 128), jnp.float32),
            pltpu.VMEM((N_DEV, 128), jnp.float32),
            pltpu.SemaphoreType.DMA,
            pltpu.SemaphoreType.DMA((N_DEV,)),
            pltpu.SemaphoreType.DMA((N_DEV,)),
            pltpu.SemaphoreType.DMA((N_DEV,)),
            pltpu.SemaphoreType.DMA((N_DEV,)),
        ],
        compiler_params=pltpu.CompilerParams(
            vmem_limit_bytes=60 * 1024 * 1024,
            collective_id=0,
        ),
    )(x, w_mat)
